# baseline (speedup 1.0000x reference)
import functools

import jax
import jax.numpy as jnp
from jax import lax
from jax.experimental import pallas as pl
from jax.experimental.pallas import tpu as pltpu
from jax.experimental.pallas import tpu_sc as plsc

N = 55296
E = 442368
DIN = 256
DH = 512
B = 1024
G = 64
A = 18
NODES = 54

NC = 2
NS = 16
L = 16

F32 = jnp.float32
I32 = jnp.int32

C_PER_SC = 4
NCHUNK = NC * C_PER_SC
V = N // NCHUNK
SLAB_ROWS = V + L
EPT = E // NS
SB = 6912
NSB = EPT // SB
CAP_ROWS = SB // 128
RB = 128

MAGIC27 = 38837
MSHIFT = 20


def _iota16():
  return lax.iota(I32, L)


def _bi(s):
  return jnp.full((L,), s, I32)


def _sc1_body(x_hbm, src_hbm, dst_hbm, ew_hbm, agg_hbm,
              src_v, dst_v, ew_v, csrc, cdst, cew, rows_v, rows_w, slab,
              gsem, gsem2):
  core = lax.axis_index("c")
  tid = lax.axis_index("s")
  iota = _iota16()
  zeros32 = jnp.zeros((2 * L,), jnp.bfloat16)

  zrows_per_tile = V // NS
  for p in range(C_PER_SC):
    chunk = core * C_PER_SC + p
    lo = chunk * V

    def _z(r, _):
      for s in range(2):
        for k in range(128 // (2 * L)):
          rows_v[r, s, pl.ds(k * 2 * L, 2 * L)] = zeros32
      return 0
    lax.fori_loop(0, RB, _z, 0)

    zbase = pl.multiple_of(tid * zrows_per_tile, 16)
    nfull = zrows_per_tile // RB
    for zi in range(nfull):
      pltpu.sync_copy(rows_v, slab.at[pl.ds(zbase + zi * RB, RB)])
    rem = zrows_per_tile - nfull * RB
    if rem:
      pltpu.sync_copy(rows_v.at[pl.ds(0, rem)],
                      slab.at[pl.ds(zbase + nfull * RB, rem)])
    @pl.when(tid == 0)
    def _():
      pltpu.sync_copy(rows_v.at[pl.ds(0, L)], slab.at[pl.ds(V, L)])
    plsc.subcore_barrier()

    for sb in range(NSB):
      ebase = pl.multiple_of(tid * EPT + sb * SB, 128)
      pltpu.sync_copy(src_hbm.at[pl.ds(ebase, SB)], src_v)
      pltpu.sync_copy(dst_hbm.at[pl.ds(ebase, SB)], dst_v)
      pltpu.sync_copy(ew_hbm.at[pl.ds(ebase, SB)], ew_v)

      def _scan(i, cnt):
        off = i * L
        d = dst_v[pl.ds(off, L)]
        ld = d - _bi(lo)
        m = (ld >= _bi(0)) & (ld < _bi(V))
        cs = plsc.cumsum(m.astype(I32))
        pos = _bi(cnt) + cs - _bi(1)
        row = lax.shift_right_logical(pos, _bi(7))
        col = pos & _bi(127)
        plsc.store_scatter(cdst, [row, col], ld, mask=m)
        plsc.store_scatter(csrc, [row, col], src_v[pl.ds(off, L)], mask=m)
        plsc.store_scatter(cew, [row, col], ew_v[pl.ds(off, L)], mask=m)
        return cnt + jnp.sum(m.astype(I32))

      cnt = lax.fori_loop(0, SB // L, _scan, jnp.int32(0))
      nb = lax.shift_right_logical(cnt + 127, 7)
      lim = lax.shift_left(nb, 7)
      for k in range(RB // L):
        posp = _bi(cnt + k * L) + iota
        mp = posp < _bi(lim)
        rowp = lax.shift_right_logical(posp, _bi(7))
        colp = posp & _bi(127)
        plsc.store_scatter(cdst, [rowp, colp], _bi(V) + iota, mask=mp)
        plsc.store_scatter(csrc, [rowp, colp], iota, mask=mp)
        plsc.store_scatter(cew, [rowp, colp], jnp.zeros((L,), F32), mask=mp)

      def _scale_scatter(j, buf):
        def _mul(r, _):
          ewv = plsc.load_gather(
              cew, [jnp.full((L,), j, I32), jnp.full((L,), r, I32)])
          ew32 = plsc.pack(ewv, ewv, format=plsc.PackFormat.INTERLEAVED)
          for s in range(2):
            for k in range(128 // (2 * L)):
              sl = pl.ds(k * 2 * L, 2 * L)
              buf[r, s, sl] = buf[r, s, sl] * ew32
          return 0
        lax.fori_loop(0, RB, _mul, 0)
        pltpu.sync_copy(buf, slab.at[cdst.at[j]], add=True)

      @pl.when(nb > 0)
      def _():
        pltpu.async_copy(x_hbm.at[csrc.at[0]], rows_v, gsem)

      npair = lax.shift_right_logical(nb + 1, 1)

      def _pair(q, _):
        j0 = q * 2
        j1 = j0 + 1
        pltpu.make_async_copy(x_hbm.at[csrc.at[j0]], rows_v, gsem).wait()

        @pl.when(j1 < nb)
        def _():
          pltpu.async_copy(x_hbm.at[csrc.at[j1]], rows_w, gsem2)
        _scale_scatter(j0, rows_v)

        @pl.when(j1 < nb)
        def _():
          pltpu.make_async_copy(x_hbm.at[csrc.at[j1]], rows_w, gsem2).wait()

          @pl.when(j1 + 1 < nb)
          def _():
            pltpu.async_copy(x_hbm.at[csrc.at[j1 + 1]], rows_v, gsem)
          _scale_scatter(j1, rows_w)
        return 0

      lax.fori_loop(0, npair, _pair, 0)

    plsc.subcore_barrier()
    wrows = V // NS
    wbase = pl.multiple_of(tid * wrows, 16)
    pltpu.sync_copy(slab.at[pl.ds(wbase, wrows)],
                    agg_hbm.at[pl.ds(pl.multiple_of(lo + tid * wrows, 16),
                                     wrows)])
    plsc.subcore_barrier()


def _sc_agg1(xbf, src, dst, ew):
  mesh = plsc.VectorSubcoreMesh(core_axis_name="c", subcore_axis_name="s",
                                num_cores=NC, num_subcores=NS)
  f = pl.kernel(
      _sc1_body,
      out_type=jax.ShapeDtypeStruct((N, 2, 128), jnp.bfloat16),
      mesh=mesh,
      compiler_params=pltpu.CompilerParams(needs_layout_passes=False,
                                           use_tc_tiling_on_sc=False),
      scratch_types=[
          pltpu.VMEM((SB,), I32),
          pltpu.VMEM((SB,), I32),
          pltpu.VMEM((SB,), F32),
          pltpu.VMEM((CAP_ROWS, 128), I32),
          pltpu.VMEM((CAP_ROWS, 128), I32),
          pltpu.VMEM((CAP_ROWS, 128), F32),
          pltpu.VMEM((RB, 2, 128), jnp.bfloat16),
          pltpu.VMEM((RB, 2, 128), jnp.bfloat16),
          pltpu.VMEM_SHARED((SLAB_ROWS, 2, 128), jnp.bfloat16),
          pltpu.SemaphoreType.DMA,
          pltpu.SemaphoreType.DMA,
      ],
  )
  return f(xbf, src, dst, ew)


GROW = 56
SLAB2_ROWS = B * GROW
EPT2 = E // (NC * NS)
EB2 = 512
NB2 = EPT2 // EB2
NPT = N // (NC * NS)


def _row56(d):
  t = lax.shift_right_logical(d, _bi(1))
  b = lax.shift_right_logical(t * _bi(MAGIC27), _bi(MSHIFT))
  return d + b + b


def _sc2_body(pdup_hbm, qdup_hbm, src_hbm, dst_hbm, ew_hbm, out_hbm,
              srcb, dstb, ewb, rb, rows2, qbuf, qri, zbuf2, slab2, gsem):
  core = lax.axis_index("c")
  tid = lax.axis_index("s")
  w = core * NS + tid
  iota = _iota16()

  def _z(r, _):
    flat = _bi(r * L) + iota
    plsc.store_scatter(zbuf2,
                       [lax.shift_right_logical(flat, _bi(2)), iota & _bi(3)],
                       jnp.zeros((L,), F32))
    return 0
  lax.fori_loop(0, 128, _z, 0)

  zbase = pl.multiple_of(tid * (SLAB2_ROWS // NS), 128)
  for zi in range(7):
    pltpu.sync_copy(zbuf2, slab2.at[pl.ds(zbase + zi * 512, 512)])
  plsc.subcore_barrier()

  nbase = w * NPT
  pltpu.sync_copy(qdup_hbm.at[pl.ds(pl.multiple_of(nbase, 64), NPT)], qbuf)

  def _qr(i, _):
    d = _bi(nbase + i * L) + iota
    qri[pl.ds(i * L, L)] = _row56(d)
    return 0
  lax.fori_loop(0, NPT // L, _qr, 0)
  pltpu.sync_copy(qbuf, slab2.at[qri], add=True)

  def _batch(j, _):
    ebase = pl.multiple_of(w * EPT2 + j * EB2, 128)
    pltpu.sync_copy(src_hbm.at[pl.ds(ebase, EB2)], srcb)
    pltpu.sync_copy(dst_hbm.at[pl.ds(ebase, EB2)], dstb)
    pltpu.sync_copy(ew_hbm.at[pl.ds(ebase, EB2)], ewb)
    pltpu.async_copy(pdup_hbm.at[srcb], rows2, gsem).wait()

    def _ri(i, _):
      d = dstb[pl.ds(i * L, L)]
      rb[pl.ds(i * L, L)] = _row56(d)
      return 0
    lax.fori_loop(0, EB2 // L, _ri, 0)

    def _mul(i, _):
      rowi = _bi(i * 4) + lax.shift_right_logical(iota, _bi(2))
      coli = iota & _bi(3)
      ewv = plsc.load_gather(ewb, [rowi])
      v = plsc.load_gather(rows2, [rowi, coli])
      plsc.store_scatter(rows2, [rowi, coli], v * ewv)
      return 0
    lax.fori_loop(0, EB2 // 4, _mul, 0)
    pltpu.sync_copy(rows2, slab2.at[rb], add=True)
    return 0

  lax.fori_loop(0, NB2, _batch, 0)

  plsc.subcore_barrier()
  wrows = SLAB2_ROWS // NS
  wbase = pl.multiple_of(tid * wrows, 128)
  pltpu.sync_copy(slab2.at[pl.ds(wbase, wrows)],
                  out_hbm.at[core, pl.ds(wbase, wrows)])


def _sc_agg2(pdup, qdup, src, dst, ew):
  mesh = plsc.VectorSubcoreMesh(core_axis_name="c", subcore_axis_name="s",
                                num_cores=NC, num_subcores=NS)
  f = pl.kernel(
      _sc2_body,
      out_type=jax.ShapeDtypeStruct((NC, SLAB2_ROWS, 4), F32),
      mesh=mesh,
      compiler_params=pltpu.CompilerParams(needs_layout_passes=False,
                                           use_tc_tiling_on_sc=False),
      scratch_types=[
          pltpu.VMEM((EB2,), I32),
          pltpu.VMEM((EB2,), I32),
          pltpu.VMEM((EB2,), F32),
          pltpu.VMEM((EB2,), I32),
          pltpu.VMEM((EB2, 4), F32),
          pltpu.VMEM((NPT, 4), F32),
          pltpu.VMEM((NPT,), I32),
          pltpu.VMEM((512, 4), F32),
          pltpu.VMEM_SHARED((SLAB2_ROWS, 4), F32),
          pltpu.SemaphoreType.DMA,
      ],
  )
  return f(pdup, qdup, src, dst, ew)


RBA = 432


def _tca_body(x_ref, agg_ref, wrel1a_ref, wrel1b_ref, wroot1a_ref,
              wroot1b_ref, brel1_ref, wpd_ref, wqd_ref, pdup_ref, qdup_ref):
  dn = (((1,), (1,)), ((), ()))
  a0 = agg_ref[:, 0, :]
  a1 = agg_ref[:, 1, :]
  x0 = x_ref[:, 0, :]
  x1 = x_ref[:, 1, :]
  h = lax.dot_general(a0, wrel1a_ref[...], dn, preferred_element_type=F32)
  h += lax.dot_general(a1, wrel1b_ref[...], dn, preferred_element_type=F32)
  h += lax.dot_general(x0, wroot1a_ref[...], dn, preferred_element_type=F32)
  h += lax.dot_general(x1, wroot1b_ref[...], dn, preferred_element_type=F32)
  h += brel1_ref[...]
  h = jnp.maximum(h, 0.0).astype(jnp.bfloat16)
  pdup_ref[...] = lax.dot_general(h, wpd_ref[...], dn,
                                  preferred_element_type=F32)
  qdup_ref[...] = lax.dot_general(h, wqd_ref[...], dn,
                                  preferred_element_type=F32)


def _tc_layer1(xbf, agg1, Wrel1, brel1, Wroot1, Wrel2, Wroot2):
  bf = jnp.bfloat16
  wpd = Wrel2.astype(bf)
  wqd = Wroot2.astype(bf)
  grid = (N // RBA,)
  return pl.pallas_call(
      _tca_body,
      grid=grid,
      in_specs=[
          pl.BlockSpec((RBA, 2, 128), lambda i: (i, 0, 0)),
          pl.BlockSpec((RBA, 2, 128), lambda i: (i, 0, 0)),
          pl.BlockSpec((DH, 128), lambda i: (0, 0)),
          pl.BlockSpec((DH, 128), lambda i: (0, 0)),
          pl.BlockSpec((DH, 128), lambda i: (0, 0)),
          pl.BlockSpec((DH, 128), lambda i: (0, 0)),
          pl.BlockSpec((1, DH), lambda i: (0, 0)),
          pl.BlockSpec((4, DH), lambda i: (0, 0)),
          pl.BlockSpec((4, DH), lambda i: (0, 0)),
      ],
      out_specs=[
          pl.BlockSpec((RBA, 4), lambda i: (i, 0)),
          pl.BlockSpec((RBA, 4), lambda i: (i, 0)),
      ],
      out_shape=[
          jax.ShapeDtypeStruct((N, 4), F32),
          jax.ShapeDtypeStruct((N, 4), F32),
      ],
  )(xbf, agg1, Wrel1[:, :128].astype(bf), Wrel1[:, 128:].astype(bf),
    Wroot1[:, :128].astype(bf), Wroot1[:, 128:].astype(bf),
    brel1.reshape(1, DH), wpd, wqd)


GBLK = 256
E224 = GROW * 4


def _tcb_body(s_ref, glob_ref, brel2t_ref, wg1_ref, bg1_ref, wg2_ref,
              bg2_ref, wg3_ref, bg3_ref, wo1e_ref, wo1g_ref, bo1_ref,
              wo2_ref, bo2_ref, wo3_ref, bo3_ref, out_ref):
  dn = (((1,), (1,)), ((), ()))
  e = jnp.maximum(s_ref[0] + s_ref[1] + brel2t_ref[...], 0.0)
  g = jnp.maximum(lax.dot_general(glob_ref[...], wg1_ref[...], dn,
                                  preferred_element_type=F32)
                  + bg1_ref[...], 0.0)
  g = jnp.maximum(lax.dot_general(g, wg2_ref[...], dn,
                                  preferred_element_type=F32)
                  + bg2_ref[...], 0.0)
  g = jnp.maximum(lax.dot_general(g, wg3_ref[...], dn,
                                  preferred_element_type=F32)
                  + bg3_ref[...], 0.0)
  o = lax.dot_general(e, wo1e_ref[...], dn, preferred_element_type=F32)
  o += lax.dot_general(g, wo1g_ref[...], dn, preferred_element_type=F32)
  o = jnp.maximum(o + bo1_ref[...], 0.0)
  o = jnp.maximum(lax.dot_general(o, wo2_ref[...], dn,
                                  preferred_element_type=F32)
                  + bo2_ref[...], 0.0)
  out_ref[...] = (lax.dot_general(o, wo3_ref[...], dn,
                                  preferred_element_type=F32)
                  + bo3_ref[...])


def _tc_head(slabs, glob, brel2, Wg1, bg1, Wg2, bg2, Wg3, bg3,
             Wo1, bo1, Wo2, bo2, Wo3, bo3):
  brel2t = jnp.concatenate(
      [jnp.tile(brel2, NODES), jnp.zeros((8,), F32)]).reshape(1, E224)
  wo1e = jnp.concatenate(
      [Wo1[:, :NODES * 4], jnp.zeros((128, 8), F32)], axis=1)
  wo1g = Wo1[:, NODES * 4:]
  grid = (B // GBLK,)
  full = lambda shape: pl.BlockSpec(shape, lambda i: tuple(0 for _ in shape))
  return pl.pallas_call(
      _tcb_body,
      grid=grid,
      in_specs=[
          pl.BlockSpec((NC, GBLK, E224), lambda i: (0, i, 0)),
          pl.BlockSpec((GBLK, G), lambda i: (i, 0)),
          full((1, E224)),
          full((8, G)), full((1, 8)),
          full((8, 8)), full((1, 8)),
          full((G, 8)), full((1, G)),
          full((128, E224)), full((128, G)), full((1, 128)),
          full((128, 128)), full((1, 128)),
          full((A, 128)), full((1, A)),
      ],
      out_specs=pl.BlockSpec((GBLK, A), lambda i: (i, 0)),
      out_shape=jax.ShapeDtypeStruct((B, A), F32),
  )(slabs, glob, brel2t, Wg1, bg1.reshape(1, 8), Wg2, bg2.reshape(1, 8),
    Wg3, bg3.reshape(1, G), wo1e, wo1g, bo1.reshape(1, 128),
    Wo2, bo2.reshape(1, 128), Wo3, bo3.reshape(1, A))


def kernel(x, edge_index, edge_attr, glob, Wrel1, brel1, Wroot1, Wrel2,
           brel2, Wroot2, Wg1, bg1, Wg2, bg2, Wg3, bg3, Wo1, bo1, Wo2, bo2,
           Wo3, bo3):
  src = edge_index[0]
  dst = edge_index[1]
  xbf = x.astype(jnp.bfloat16).reshape(N, 2, 128)
  agg1 = _sc_agg1(xbf, src, dst, edge_attr)
  pdup, qdup = _tc_layer1(xbf, agg1, Wrel1, brel1, Wroot1, Wrel2, Wroot2)
  agg2 = _sc_agg2(pdup, qdup, src, dst, edge_attr)
  slabs = agg2.reshape(NC, B, E224)
  return _tc_head(slabs, glob, brel2, Wg1, bg1, Wg2, bg2, Wg3, bg3,
                  Wo1, bo1, Wo2, bo2, Wo3, bo3)


# speedup vs baseline: 1.0736x; 1.0736x over previous
import functools

import jax
import jax.numpy as jnp
from jax import lax
from jax.experimental import pallas as pl
from jax.experimental.pallas import tpu as pltpu
from jax.experimental.pallas import tpu_sc as plsc

N = 55296
E = 442368
DIN = 256
DH = 512
B = 1024
G = 64
A = 18
NODES = 54

NC = 2
NS = 16
L = 16

F32 = jnp.float32
I32 = jnp.int32

C_PER_SC = 4
NCHUNK = NC * C_PER_SC
V = N // NCHUNK
SLAB_ROWS = V + L
EPT = E // NS
SB = 6912
NSB = EPT // SB
CAP_ROWS = SB // 128
RB = 128

MAGIC27 = 38837
MSHIFT = 20


def _iota16():
  return lax.iota(I32, L)


def _bi(s):
  return jnp.full((L,), s, I32)


def _sc1_body(x_hbm, src_hbm, dst_hbm, ew_hbm, agg_hbm,
              src_v, dst_v, ew_v, csrc, cdst, cew, rows_v, rows_w, slab,
              gsem, gsem2):
  core = lax.axis_index("c")
  tid = lax.axis_index("s")
  iota = _iota16()
  zeros32 = jnp.zeros((2 * L,), jnp.bfloat16)

  zrows_per_tile = V // NS
  for p in range(C_PER_SC):
    chunk = core * C_PER_SC + p
    lo = chunk * V

    def _z(r, _):
      for s in range(2):
        for k in range(128 // (2 * L)):
          rows_v[r, s, pl.ds(k * 2 * L, 2 * L)] = zeros32
      return 0
    lax.fori_loop(0, RB, _z, 0)

    zbase = pl.multiple_of(tid * zrows_per_tile, 16)
    nfull = zrows_per_tile // RB
    for zi in range(nfull):
      pltpu.sync_copy(rows_v, slab.at[pl.ds(zbase + zi * RB, RB)])
    rem = zrows_per_tile - nfull * RB
    if rem:
      pltpu.sync_copy(rows_v.at[pl.ds(0, rem)],
                      slab.at[pl.ds(zbase + nfull * RB, rem)])
    @pl.when(tid == 0)
    def _():
      pltpu.sync_copy(rows_v.at[pl.ds(0, L)], slab.at[pl.ds(V, L)])
    plsc.subcore_barrier()

    for sb in range(NSB):
      ebase = pl.multiple_of(tid * EPT + sb * SB, 128)
      pltpu.sync_copy(src_hbm.at[pl.ds(ebase, SB)], src_v)
      pltpu.sync_copy(dst_hbm.at[pl.ds(ebase, SB)], dst_v)
      pltpu.sync_copy(ew_hbm.at[pl.ds(ebase, SB)], ew_v)

      def _scan(i, cnt):
        off = i * L
        d = dst_v[pl.ds(off, L)]
        ld = d - _bi(lo)
        m = (ld >= _bi(0)) & (ld < _bi(V))
        cs = plsc.cumsum(m.astype(I32))
        pos = _bi(cnt) + cs - _bi(1)
        row = lax.shift_right_logical(pos, _bi(7))
        col = pos & _bi(127)
        plsc.store_scatter(cdst, [row, col], ld, mask=m)
        plsc.store_scatter(csrc, [row, col], src_v[pl.ds(off, L)], mask=m)
        plsc.store_scatter(cew, [row, col], ew_v[pl.ds(off, L)], mask=m)
        return cnt + jnp.sum(m.astype(I32))

      cnt = lax.fori_loop(0, SB // L, _scan, jnp.int32(0))
      nb = lax.shift_right_logical(cnt + 127, 7)
      lim = lax.shift_left(nb, 7)
      for k in range(RB // L):
        posp = _bi(cnt + k * L) + iota
        mp = posp < _bi(lim)
        rowp = lax.shift_right_logical(posp, _bi(7))
        colp = posp & _bi(127)
        plsc.store_scatter(cdst, [rowp, colp], _bi(V) + iota, mask=mp)
        plsc.store_scatter(csrc, [rowp, colp], iota, mask=mp)
        plsc.store_scatter(cew, [rowp, colp], jnp.zeros((L,), F32), mask=mp)

      def _scale_scatter(j, buf):
        jv = jnp.full((L,), j, I32)

        def _mul(r2, _):
          r = r2 * 2
          for dr in range(2):
            ewv = plsc.load_gather(cew, [jv, _bi(r + dr)])
            ew32 = plsc.pack(ewv, ewv, format=plsc.PackFormat.INTERLEAVED)
            for s in range(2):
              for k in range(128 // (2 * L)):
                sl = pl.ds(k * 2 * L, 2 * L)
                buf[r + dr, s, sl] = buf[r + dr, s, sl] * ew32
          return 0
        lax.fori_loop(0, RB // 2, _mul, 0)
        pltpu.sync_copy(buf, slab.at[cdst.at[j]], add=True)

      @pl.when(nb > 0)
      def _():
        pltpu.async_copy(x_hbm.at[csrc.at[0]], rows_v, gsem)

      npair = lax.shift_right_logical(nb + 1, 1)

      def _pair(q, _):
        j0 = q * 2
        j1 = j0 + 1
        pltpu.make_async_copy(x_hbm.at[csrc.at[j0]], rows_v, gsem).wait()

        @pl.when(j1 < nb)
        def _():
          pltpu.async_copy(x_hbm.at[csrc.at[j1]], rows_w, gsem2)
        _scale_scatter(j0, rows_v)

        @pl.when(j1 < nb)
        def _():
          pltpu.make_async_copy(x_hbm.at[csrc.at[j1]], rows_w, gsem2).wait()

          @pl.when(j1 + 1 < nb)
          def _():
            pltpu.async_copy(x_hbm.at[csrc.at[j1 + 1]], rows_v, gsem)
          _scale_scatter(j1, rows_w)
        return 0

      lax.fori_loop(0, npair, _pair, 0)

    plsc.subcore_barrier()
    wrows = V // NS
    wbase = pl.multiple_of(tid * wrows, 16)
    pltpu.sync_copy(slab.at[pl.ds(wbase, wrows)],
                    agg_hbm.at[pl.ds(pl.multiple_of(lo + tid * wrows, 16),
                                     wrows)])
    plsc.subcore_barrier()


def _sc_agg1(xbf, src, dst, ew):
  mesh = plsc.VectorSubcoreMesh(core_axis_name="c", subcore_axis_name="s",
                                num_cores=NC, num_subcores=NS)
  f = pl.kernel(
      _sc1_body,
      out_type=jax.ShapeDtypeStruct((N, 2, 128), jnp.bfloat16),
      mesh=mesh,
      compiler_params=pltpu.CompilerParams(needs_layout_passes=False,
                                           use_tc_tiling_on_sc=False),
      scratch_types=[
          pltpu.VMEM((SB,), I32),
          pltpu.VMEM((SB,), I32),
          pltpu.VMEM((SB,), F32),
          pltpu.VMEM((CAP_ROWS, 128), I32),
          pltpu.VMEM((CAP_ROWS, 128), I32),
          pltpu.VMEM((CAP_ROWS, 128), F32),
          pltpu.VMEM((RB, 2, 128), jnp.bfloat16),
          pltpu.VMEM((RB, 2, 128), jnp.bfloat16),
          pltpu.VMEM_SHARED((SLAB_ROWS, 2, 128), jnp.bfloat16),
          pltpu.SemaphoreType.DMA,
          pltpu.SemaphoreType.DMA,
      ],
  )
  return f(xbf, src, dst, ew)


GROW = 28
SLAB2_ROWS = B * GROW
EPT2 = E // (NC * NS)
EB2 = 512
NB2 = EPT2 // EB2
NPT = N // (NC * NS)


def _sc2_body(pdup_hbm, qdup_hbm, src_hbm, dst_hbm, ew_hbm, out_hbm,
              srcb, dstb, ewb, rb, rows2, qbuf, qri, zbuf2, slab2, gsem):
  core = lax.axis_index("c")
  tid = lax.axis_index("s")
  w = core * NS + tid
  iota = _iota16()
  halfsel = lax.shift_right_logical(iota, _bi(2)) & _bi(1)
  qmask = ((halfsel == (lax.shift_right_logical(iota, _bi(3)) & _bi(1)))
           .astype(F32))

  def _z(r, _):
    flat = _bi(r * L) + iota
    plsc.store_scatter(zbuf2,
                       [lax.shift_right_logical(flat, _bi(3)), iota & _bi(7)],
                       jnp.zeros((L,), F32))
    return 0
  lax.fori_loop(0, 128, _z, 0)

  zbase = pl.multiple_of(tid * (SLAB2_ROWS // NS), 128)
  for zi in range(7):
    pltpu.sync_copy(zbuf2, slab2.at[pl.ds(zbase + zi * 256, 256)])
  plsc.subcore_barrier()

  nbase = w * NPT
  pltpu.sync_copy(qdup_hbm.at[pl.ds(pl.multiple_of(nbase, 64), NPT)], qbuf)

  def _qm(i, _):
    row = _bi(i * 2) + lax.shift_right_logical(iota, _bi(3))
    col = iota & _bi(7)
    v = plsc.load_gather(qbuf, [row, col])
    plsc.store_scatter(qbuf, [row, col], v * qmask)
    return 0

  def _qr(i, _):
    d = _bi(nbase + i * L) + iota
    t = lax.shift_right_logical(d, _bi(1))
    r = t + lax.shift_right_logical(t * _bi(MAGIC27), _bi(MSHIFT))
    qri[pl.ds(i * L, L)] = r
    return 0
  lax.fori_loop(0, (NPT * 8) // L, _qm, 0)
  lax.fori_loop(0, NPT // L, _qr, 0)
  pltpu.sync_copy(qbuf, slab2.at[qri], add=True)

  def _batch(j, _):
    ebase = pl.multiple_of(w * EPT2 + j * EB2, 128)
    pltpu.sync_copy(src_hbm.at[pl.ds(ebase, EB2)], srcb)
    pltpu.sync_copy(dst_hbm.at[pl.ds(ebase, EB2)], dstb)
    pltpu.sync_copy(ew_hbm.at[pl.ds(ebase, EB2)], ewb)
    pltpu.async_copy(pdup_hbm.at[srcb], rows2, gsem).wait()

    def _ri(i, _):
      d = dstb[pl.ds(i * L, L)]
      t = lax.shift_right_logical(d, _bi(1))
      rb[pl.ds(i * L, L)] = t + lax.shift_right_logical(
          t * _bi(MAGIC27), _bi(MSHIFT))
      return 0
    lax.fori_loop(0, EB2 // L, _ri, 0)

    def _mul(i, _):
      eidx = _bi(i * 2) + lax.shift_right_logical(iota, _bi(3))
      ewv = plsc.load_gather(ewb, [eidx])
      dv = plsc.load_gather(dstb, [eidx])
      keep = (halfsel == (dv & _bi(1))).astype(F32)
      rowi = eidx
      coli = iota & _bi(7)
      v = plsc.load_gather(rows2, [rowi, coli])
      plsc.store_scatter(rows2, [rowi, coli], v * ewv * keep)
      return 0
    lax.fori_loop(0, EB2 // 2, _mul, 0)
    pltpu.sync_copy(rows2, slab2.at[rb], add=True)
    return 0

  lax.fori_loop(0, NB2, _batch, 0)

  plsc.subcore_barrier()
  wrows = SLAB2_ROWS // NS
  wbase = pl.multiple_of(tid * wrows, 128)
  pltpu.sync_copy(slab2.at[pl.ds(wbase, wrows)],
                  out_hbm.at[core, pl.ds(wbase, wrows)])


def _sc_agg2(pdup, qdup, src, dst, ew):
  mesh = plsc.VectorSubcoreMesh(core_axis_name="c", subcore_axis_name="s",
                                num_cores=NC, num_subcores=NS)
  f = pl.kernel(
      _sc2_body,
      out_type=jax.ShapeDtypeStruct((NC, SLAB2_ROWS, 8), F32),
      mesh=mesh,
      compiler_params=pltpu.CompilerParams(needs_layout_passes=False,
                                           use_tc_tiling_on_sc=False),
      scratch_types=[
          pltpu.VMEM((EB2,), I32),
          pltpu.VMEM((EB2,), I32),
          pltpu.VMEM((EB2,), F32),
          pltpu.VMEM((EB2,), I32),
          pltpu.VMEM((EB2, 8), F32),
          pltpu.VMEM((NPT, 8), F32),
          pltpu.VMEM((NPT,), I32),
          pltpu.VMEM((256, 8), F32),
          pltpu.VMEM_SHARED((SLAB2_ROWS, 8), F32),
          pltpu.SemaphoreType.DMA,
      ],
  )
  return f(pdup, qdup, src, dst, ew)


RBA = 432


def _tca_body(x_ref, agg_ref, wrel1a_ref, wrel1b_ref, wroot1a_ref,
              wroot1b_ref, brel1_ref, wpd_ref, wqd_ref, pdup_ref, qdup_ref):
  dn = (((1,), (1,)), ((), ()))
  a0 = agg_ref[:, 0, :]
  a1 = agg_ref[:, 1, :]
  x0 = x_ref[:, 0, :]
  x1 = x_ref[:, 1, :]
  h = lax.dot_general(a0, wrel1a_ref[...], dn, preferred_element_type=F32)
  h += lax.dot_general(a1, wrel1b_ref[...], dn, preferred_element_type=F32)
  h += lax.dot_general(x0, wroot1a_ref[...], dn, preferred_element_type=F32)
  h += lax.dot_general(x1, wroot1b_ref[...], dn, preferred_element_type=F32)
  h += brel1_ref[...]
  h = jnp.maximum(h, 0.0).astype(jnp.bfloat16)
  pdup_ref[...] = lax.dot_general(h, wpd_ref[...], dn,
                                  preferred_element_type=F32)
  qdup_ref[...] = lax.dot_general(h, wqd_ref[...], dn,
                                  preferred_element_type=F32)


def _tc_layer1(xbf, agg1, Wrel1, brel1, Wroot1, Wrel2, Wroot2):
  bf = jnp.bfloat16
  wpd = jnp.concatenate([Wrel2, Wrel2], axis=0).astype(bf)
  wqd = jnp.concatenate([Wroot2, Wroot2], axis=0).astype(bf)
  grid = (N // RBA,)
  return pl.pallas_call(
      _tca_body,
      grid=grid,
      in_specs=[
          pl.BlockSpec((RBA, 2, 128), lambda i: (i, 0, 0)),
          pl.BlockSpec((RBA, 2, 128), lambda i: (i, 0, 0)),
          pl.BlockSpec((DH, 128), lambda i: (0, 0)),
          pl.BlockSpec((DH, 128), lambda i: (0, 0)),
          pl.BlockSpec((DH, 128), lambda i: (0, 0)),
          pl.BlockSpec((DH, 128), lambda i: (0, 0)),
          pl.BlockSpec((1, DH), lambda i: (0, 0)),
          pl.BlockSpec((8, DH), lambda i: (0, 0)),
          pl.BlockSpec((8, DH), lambda i: (0, 0)),
      ],
      out_specs=[
          pl.BlockSpec((RBA, 8), lambda i: (i, 0)),
          pl.BlockSpec((RBA, 8), lambda i: (i, 0)),
      ],
      out_shape=[
          jax.ShapeDtypeStruct((N, 8), F32),
          jax.ShapeDtypeStruct((N, 8), F32),
      ],
  )(xbf, agg1, Wrel1[:, :128].astype(bf), Wrel1[:, 128:].astype(bf),
    Wroot1[:, :128].astype(bf), Wroot1[:, 128:].astype(bf),
    brel1.reshape(1, DH), wpd, wqd)


GBLK = 256
E224 = GROW * 8


def _tcb_body(s_ref, glob_ref, brel2t_ref, wg1_ref, bg1_ref, wg2_ref,
              bg2_ref, wg3_ref, bg3_ref, wo1e_ref, wo1g_ref, bo1_ref,
              wo2_ref, bo2_ref, wo3_ref, bo3_ref, out_ref):
  dn = (((1,), (1,)), ((), ()))
  e = jnp.maximum(s_ref[0] + s_ref[1] + brel2t_ref[...], 0.0)
  g = jnp.maximum(lax.dot_general(glob_ref[...], wg1_ref[...], dn,
                                  preferred_element_type=F32)
                  + bg1_ref[...], 0.0)
  g = jnp.maximum(lax.dot_general(g, wg2_ref[...], dn,
                                  preferred_element_type=F32)
                  + bg2_ref[...], 0.0)
  g = jnp.maximum(lax.dot_general(g, wg3_ref[...], dn,
                                  preferred_element_type=F32)
                  + bg3_ref[...], 0.0)
  o = lax.dot_general(e, wo1e_ref[...], dn, preferred_element_type=F32)
  o += lax.dot_general(g, wo1g_ref[...], dn, preferred_element_type=F32)
  o = jnp.maximum(o + bo1_ref[...], 0.0)
  o = jnp.maximum(lax.dot_general(o, wo2_ref[...], dn,
                                  preferred_element_type=F32)
                  + bo2_ref[...], 0.0)
  out_ref[...] = (lax.dot_general(o, wo3_ref[...], dn,
                                  preferred_element_type=F32)
                  + bo3_ref[...])


def _tc_head(slabs, glob, brel2, Wg1, bg1, Wg2, bg2, Wg3, bg3,
             Wo1, bo1, Wo2, bo2, Wo3, bo3):
  brel2t = jnp.concatenate(
      [jnp.tile(brel2, NODES), jnp.zeros((8,), F32)]).reshape(1, E224)
  wo1e = jnp.concatenate(
      [Wo1[:, :NODES * 4], jnp.zeros((128, 8), F32)], axis=1)
  wo1g = Wo1[:, NODES * 4:]
  grid = (B // GBLK,)
  full = lambda shape: pl.BlockSpec(shape, lambda i: tuple(0 for _ in shape))
  return pl.pallas_call(
      _tcb_body,
      grid=grid,
      in_specs=[
          pl.BlockSpec((NC, GBLK, E224), lambda i: (0, i, 0)),
          pl.BlockSpec((GBLK, G), lambda i: (i, 0)),
          full((1, E224)),
          full((8, G)), full((1, 8)),
          full((8, 8)), full((1, 8)),
          full((G, 8)), full((1, G)),
          full((128, E224)), full((128, G)), full((1, 128)),
          full((128, 128)), full((1, 128)),
          full((A, 128)), full((1, A)),
      ],
      out_specs=pl.BlockSpec((GBLK, A), lambda i: (i, 0)),
      out_shape=jax.ShapeDtypeStruct((B, A), F32),
  )(slabs, glob, brel2t, Wg1, bg1.reshape(1, 8), Wg2, bg2.reshape(1, 8),
    Wg3, bg3.reshape(1, G), wo1e, wo1g, bo1.reshape(1, 128),
    Wo2, bo2.reshape(1, 128), Wo3, bo3.reshape(1, A))


def kernel(x, edge_index, edge_attr, glob, Wrel1, brel1, Wroot1, Wrel2,
           brel2, Wroot2, Wg1, bg1, Wg2, bg2, Wg3, bg3, Wo1, bo1, Wo2, bo2,
           Wo3, bo3):
  src = edge_index[0]
  dst = edge_index[1]
  xbf = x.astype(jnp.bfloat16).reshape(N, 2, 128)
  agg1 = _sc_agg1(xbf, src, dst, edge_attr)
  pdup, qdup = _tc_layer1(xbf, agg1, Wrel1, brel1, Wroot1, Wrel2, Wroot2)
  agg2 = _sc_agg2(pdup, qdup, src, dst, edge_attr)
  slabs = agg2.reshape(NC, B, E224)
  return _tc_head(slabs, glob, brel2, Wg1, bg1, Wg2, bg2, Wg3, bg3,
                  Wo1, bo1, Wo2, bo2, Wo3, bo3)


# speedup vs baseline: 1.1652x; 1.0853x over previous
import functools

import jax
import jax.numpy as jnp
from jax import lax
from jax.experimental import pallas as pl
from jax.experimental.pallas import tpu as pltpu
from jax.experimental.pallas import tpu_sc as plsc

N = 55296
E = 442368
DIN = 256
DH = 512
B = 1024
G = 64
A = 18
NODES = 54

NC = 2
NS = 16
L = 16

F32 = jnp.float32
I32 = jnp.int32

C_PER_SC = 4
NCHUNK = NC * C_PER_SC
V = N // NCHUNK
SLAB_ROWS = V + L
EPT = E // NS
SB = 6912
NSB = EPT // SB
CAP_ROWS = SB // 128
RB = 128

MAGIC27 = 38837
MSHIFT = 20


def _iota16():
  return lax.iota(I32, L)


def _bi(s):
  return jnp.full((L,), s, I32)


def _sc1_body(x_hbm, src_hbm, dst_hbm, ew_hbm, agg_hbm,
              src_v, dst_v, ew_v, csrc, cdst, cew, rows_v, rows_w, slab,
              gsem, gsem2):
  core = lax.axis_index("c")
  tid = lax.axis_index("s")
  iota = _iota16()
  zeros32 = jnp.zeros((2 * L,), jnp.bfloat16)

  zrows_per_tile = V // NS
  for p in range(C_PER_SC):
    chunk = core * C_PER_SC + p
    lo = chunk * V

    def _z(r, _):
      for k in range(DIN // (2 * L)):
        rows_v[r, pl.ds(k * 2 * L, 2 * L)] = zeros32
      return 0
    lax.fori_loop(0, RB, _z, 0)

    zbase = pl.multiple_of(tid * zrows_per_tile, 16)
    nfull = zrows_per_tile // RB
    for zi in range(nfull):
      pltpu.sync_copy(rows_v, slab.at[pl.ds(zbase + zi * RB, RB)])
    rem = zrows_per_tile - nfull * RB
    if rem:
      pltpu.sync_copy(rows_v.at[pl.ds(0, rem)],
                      slab.at[pl.ds(zbase + nfull * RB, rem)])
    @pl.when(tid == 0)
    def _():
      pltpu.sync_copy(rows_v.at[pl.ds(0, L)], slab.at[pl.ds(V, L)])
    plsc.subcore_barrier()

    for sb in range(NSB):
      ebase = pl.multiple_of(tid * EPT + sb * SB, 128)
      pltpu.sync_copy(src_hbm.at[pl.ds(ebase, SB)], src_v)
      pltpu.sync_copy(dst_hbm.at[pl.ds(ebase, SB)], dst_v)
      pltpu.sync_copy(ew_hbm.at[pl.ds(ebase, SB)], ew_v)

      def _scan(i, cnt):
        off = i * L
        d = dst_v[pl.ds(off, L)]
        ld = d - _bi(lo)
        m = (ld >= _bi(0)) & (ld < _bi(V))
        cs = plsc.cumsum(m.astype(I32))
        pos = _bi(cnt) + cs - _bi(1)
        row = lax.shift_right_logical(pos, _bi(7))
        col = pos & _bi(127)
        plsc.store_scatter(cdst, [row, col], ld, mask=m)
        plsc.store_scatter(csrc, [row, col], src_v[pl.ds(off, L)], mask=m)
        plsc.store_scatter(cew, [row, col], ew_v[pl.ds(off, L)], mask=m)
        return cnt + jnp.sum(m.astype(I32))

      cnt = lax.fori_loop(0, SB // L, _scan, jnp.int32(0))
      nb = lax.shift_right_logical(cnt + 127, 7)
      lim = lax.shift_left(nb, 7)
      for k in range(RB // L):
        posp = _bi(cnt + k * L) + iota
        mp = posp < _bi(lim)
        rowp = lax.shift_right_logical(posp, _bi(7))
        colp = posp & _bi(127)
        plsc.store_scatter(cdst, [rowp, colp], _bi(V) + iota, mask=mp)
        plsc.store_scatter(csrc, [rowp, colp], iota, mask=mp)
        plsc.store_scatter(cew, [rowp, colp], jnp.zeros((L,), F32), mask=mp)

      def _scale_scatter(j, buf):
        jv = jnp.full((L,), j, I32)

        def _mul(r2, _):
          r = r2 * 2
          for dr in range(2):
            ewv = plsc.load_gather(cew, [jv, _bi(r + dr)])
            ew32 = plsc.pack(ewv, ewv, format=plsc.PackFormat.INTERLEAVED)
            for k in range(DIN // (2 * L)):
              sl = pl.ds(k * 2 * L, 2 * L)
              buf[r + dr, sl] = buf[r + dr, sl] * ew32
          return 0
        lax.fori_loop(0, RB // 2, _mul, 0)
        pltpu.sync_copy(buf, slab.at[cdst.at[j]], add=True)

      @pl.when(nb > 0)
      def _():
        pltpu.async_copy(x_hbm.at[csrc.at[0]], rows_v, gsem)

      npair = lax.shift_right_logical(nb + 1, 1)

      def _pair(q, _):
        j0 = q * 2
        j1 = j0 + 1
        pltpu.make_async_copy(x_hbm.at[csrc.at[j0]], rows_v, gsem).wait()

        @pl.when(j1 < nb)
        def _():
          pltpu.async_copy(x_hbm.at[csrc.at[j1]], rows_w, gsem2)
        _scale_scatter(j0, rows_v)

        @pl.when(j1 < nb)
        def _():
          pltpu.make_async_copy(x_hbm.at[csrc.at[j1]], rows_w, gsem2).wait()

          @pl.when(j1 + 1 < nb)
          def _():
            pltpu.async_copy(x_hbm.at[csrc.at[j1 + 1]], rows_v, gsem)
          _scale_scatter(j1, rows_w)
        return 0

      lax.fori_loop(0, npair, _pair, 0)

    plsc.subcore_barrier()
    wrows = V // NS
    wbase = pl.multiple_of(tid * wrows, 16)
    pltpu.sync_copy(slab.at[pl.ds(wbase, wrows)],
                    agg_hbm.at[pl.ds(pl.multiple_of(lo + tid * wrows, 16),
                                     wrows)])
    plsc.subcore_barrier()


def _sc_agg1(xbf, src, dst, ew):
  mesh = plsc.VectorSubcoreMesh(core_axis_name="c", subcore_axis_name="s",
                                num_cores=NC, num_subcores=NS)
  f = pl.kernel(
      _sc1_body,
      out_type=jax.ShapeDtypeStruct((N, DIN), jnp.bfloat16),
      mesh=mesh,
      compiler_params=pltpu.CompilerParams(needs_layout_passes=False,
                                           use_tc_tiling_on_sc=False),
      scratch_types=[
          pltpu.VMEM((SB,), I32),
          pltpu.VMEM((SB,), I32),
          pltpu.VMEM((SB,), F32),
          pltpu.VMEM((CAP_ROWS, 128), I32),
          pltpu.VMEM((CAP_ROWS, 128), I32),
          pltpu.VMEM((CAP_ROWS, 128), F32),
          pltpu.VMEM((RB, DIN), jnp.bfloat16),
          pltpu.VMEM((RB, DIN), jnp.bfloat16),
          pltpu.VMEM_SHARED((SLAB_ROWS, DIN), jnp.bfloat16),
          pltpu.SemaphoreType.DMA,
          pltpu.SemaphoreType.DMA,
      ],
  )
  return f(xbf, src, dst, ew)


GROW = 28
SLAB2_ROWS = B * GROW
EPT2 = E // (NC * NS)
EB2 = 512
NB2 = EPT2 // EB2
NPT = N // (NC * NS)


def _sc2_body(pdup_hbm, qdup_hbm, src_hbm, dst_hbm, ew_hbm, out_hbm,
              srcb, dstb, ewb, rb, rows2, qbuf, qri, zbuf2, slab2, gsem):
  core = lax.axis_index("c")
  tid = lax.axis_index("s")
  w = core * NS + tid
  iota = _iota16()
  halfsel = lax.shift_right_logical(iota, _bi(2)) & _bi(1)
  qmask = ((halfsel == (lax.shift_right_logical(iota, _bi(3)) & _bi(1)))
           .astype(F32))

  def _z(r, _):
    flat = _bi(r * L) + iota
    plsc.store_scatter(zbuf2,
                       [lax.shift_right_logical(flat, _bi(3)), iota & _bi(7)],
                       jnp.zeros((L,), F32))
    return 0
  lax.fori_loop(0, 128, _z, 0)

  zbase = pl.multiple_of(tid * (SLAB2_ROWS // NS), 128)
  for zi in range(7):
    pltpu.sync_copy(zbuf2, slab2.at[pl.ds(zbase + zi * 256, 256)])
  plsc.subcore_barrier()

  nbase = w * NPT
  pltpu.sync_copy(qdup_hbm.at[pl.ds(pl.multiple_of(nbase, 64), NPT)], qbuf)

  def _qm(i, _):
    row = _bi(i * 2) + lax.shift_right_logical(iota, _bi(3))
    col = iota & _bi(7)
    v = plsc.load_gather(qbuf, [row, col])
    plsc.store_scatter(qbuf, [row, col], v * qmask)
    return 0

  def _qr(i, _):
    d = _bi(nbase + i * L) + iota
    t = lax.shift_right_logical(d, _bi(1))
    r = t + lax.shift_right_logical(t * _bi(MAGIC27), _bi(MSHIFT))
    qri[pl.ds(i * L, L)] = r
    return 0
  lax.fori_loop(0, (NPT * 8) // L, _qm, 0)
  lax.fori_loop(0, NPT // L, _qr, 0)
  pltpu.sync_copy(qbuf, slab2.at[qri], add=True)

  def _batch(j, _):
    ebase = pl.multiple_of(w * EPT2 + j * EB2, 128)
    pltpu.sync_copy(src_hbm.at[pl.ds(ebase, EB2)], srcb)
    pltpu.sync_copy(dst_hbm.at[pl.ds(ebase, EB2)], dstb)
    pltpu.sync_copy(ew_hbm.at[pl.ds(ebase, EB2)], ewb)
    pltpu.async_copy(pdup_hbm.at[srcb], rows2, gsem).wait()

    def _ri(i, _):
      d = dstb[pl.ds(i * L, L)]
      t = lax.shift_right_logical(d, _bi(1))
      rb[pl.ds(i * L, L)] = t + lax.shift_right_logical(
          t * _bi(MAGIC27), _bi(MSHIFT))
      return 0
    lax.fori_loop(0, EB2 // L, _ri, 0)

    def _mul(i, _):
      eidx = _bi(i * 2) + lax.shift_right_logical(iota, _bi(3))
      ewv = plsc.load_gather(ewb, [eidx])
      dv = plsc.load_gather(dstb, [eidx])
      keep = (halfsel == (dv & _bi(1))).astype(F32)
      rowi = eidx
      coli = iota & _bi(7)
      v = plsc.load_gather(rows2, [rowi, coli])
      plsc.store_scatter(rows2, [rowi, coli], v * ewv * keep)
      return 0
    lax.fori_loop(0, EB2 // 2, _mul, 0)
    pltpu.sync_copy(rows2, slab2.at[rb], add=True)
    return 0

  lax.fori_loop(0, NB2, _batch, 0)

  plsc.subcore_barrier()
  wrows = SLAB2_ROWS // NS
  wbase = pl.multiple_of(tid * wrows, 128)
  pltpu.sync_copy(slab2.at[pl.ds(wbase, wrows)],
                  out_hbm.at[core, pl.ds(wbase, wrows)])


def _sc_agg2(pdup, qdup, src, dst, ew):
  mesh = plsc.VectorSubcoreMesh(core_axis_name="c", subcore_axis_name="s",
                                num_cores=NC, num_subcores=NS)
  f = pl.kernel(
      _sc2_body,
      out_type=jax.ShapeDtypeStruct((NC, SLAB2_ROWS, 8), F32),
      mesh=mesh,
      compiler_params=pltpu.CompilerParams(needs_layout_passes=False,
                                           use_tc_tiling_on_sc=False),
      scratch_types=[
          pltpu.VMEM((EB2,), I32),
          pltpu.VMEM((EB2,), I32),
          pltpu.VMEM((EB2,), F32),
          pltpu.VMEM((EB2,), I32),
          pltpu.VMEM((EB2, 8), F32),
          pltpu.VMEM((NPT, 8), F32),
          pltpu.VMEM((NPT,), I32),
          pltpu.VMEM((256, 8), F32),
          pltpu.VMEM_SHARED((SLAB2_ROWS, 8), F32),
          pltpu.SemaphoreType.DMA,
      ],
  )
  return f(pdup, qdup, src, dst, ew)


RBA = 432


def _tca_body(x_ref, agg_ref, wrel1_ref, wroot1_ref, brel1_ref,
              wpd_ref, wqd_ref, pdup_ref, qdup_ref):
  dn = (((1,), (1,)), ((), ()))
  h = lax.dot_general(agg_ref[...], wrel1_ref[...], dn,
                      preferred_element_type=F32)
  h += lax.dot_general(x_ref[...], wroot1_ref[...], dn,
                       preferred_element_type=F32)
  h += brel1_ref[...]
  h = jnp.maximum(h, 0.0).astype(jnp.bfloat16)
  pdup_ref[...] = lax.dot_general(h, wpd_ref[...], dn,
                                  preferred_element_type=F32)
  qdup_ref[...] = lax.dot_general(h, wqd_ref[...], dn,
                                  preferred_element_type=F32)


def _tc_layer1(xbf, agg1, Wrel1, brel1, Wroot1, Wrel2, Wroot2):
  bf = jnp.bfloat16
  wpd = jnp.concatenate([Wrel2, Wrel2], axis=0).astype(bf)
  wqd = jnp.concatenate([Wroot2, Wroot2], axis=0).astype(bf)
  grid = (N // RBA,)
  return pl.pallas_call(
      _tca_body,
      grid=grid,
      in_specs=[
          pl.BlockSpec((RBA, DIN), lambda i: (i, 0)),
          pl.BlockSpec((RBA, DIN), lambda i: (i, 0)),
          pl.BlockSpec((DH, DIN), lambda i: (0, 0)),
          pl.BlockSpec((DH, DIN), lambda i: (0, 0)),
          pl.BlockSpec((1, DH), lambda i: (0, 0)),
          pl.BlockSpec((8, DH), lambda i: (0, 0)),
          pl.BlockSpec((8, DH), lambda i: (0, 0)),
      ],
      out_specs=[
          pl.BlockSpec((RBA, 8), lambda i: (i, 0)),
          pl.BlockSpec((RBA, 8), lambda i: (i, 0)),
      ],
      out_shape=[
          jax.ShapeDtypeStruct((N, 8), F32),
          jax.ShapeDtypeStruct((N, 8), F32),
      ],
  )(xbf, agg1, Wrel1.astype(bf), Wroot1.astype(bf),
    brel1.reshape(1, DH), wpd, wqd)


GBLK = 256
E224 = GROW * 8


def _tcb_body(s_ref, glob_ref, brel2t_ref, wg1_ref, bg1_ref, wg2_ref,
              bg2_ref, wg3_ref, bg3_ref, wo1e_ref, wo1g_ref, bo1_ref,
              wo2_ref, bo2_ref, wo3_ref, bo3_ref, out_ref):
  dn = (((1,), (1,)), ((), ()))
  e = jnp.maximum(s_ref[0] + s_ref[1] + brel2t_ref[...], 0.0)
  g = jnp.maximum(lax.dot_general(glob_ref[...], wg1_ref[...], dn,
                                  preferred_element_type=F32)
                  + bg1_ref[...], 0.0)
  g = jnp.maximum(lax.dot_general(g, wg2_ref[...], dn,
                                  preferred_element_type=F32)
                  + bg2_ref[...], 0.0)
  g = jnp.maximum(lax.dot_general(g, wg3_ref[...], dn,
                                  preferred_element_type=F32)
                  + bg3_ref[...], 0.0)
  o = lax.dot_general(e, wo1e_ref[...], dn, preferred_element_type=F32)
  o += lax.dot_general(g, wo1g_ref[...], dn, preferred_element_type=F32)
  o = jnp.maximum(o + bo1_ref[...], 0.0)
  o = jnp.maximum(lax.dot_general(o, wo2_ref[...], dn,
                                  preferred_element_type=F32)
                  + bo2_ref[...], 0.0)
  out_ref[...] = (lax.dot_general(o, wo3_ref[...], dn,
                                  preferred_element_type=F32)
                  + bo3_ref[...])


def _tc_head(slabs, glob, brel2, Wg1, bg1, Wg2, bg2, Wg3, bg3,
             Wo1, bo1, Wo2, bo2, Wo3, bo3):
  brel2t = jnp.concatenate(
      [jnp.tile(brel2, NODES), jnp.zeros((8,), F32)]).reshape(1, E224)
  wo1e = jnp.concatenate(
      [Wo1[:, :NODES * 4], jnp.zeros((128, 8), F32)], axis=1)
  wo1g = Wo1[:, NODES * 4:]
  grid = (B // GBLK,)
  full = lambda shape: pl.BlockSpec(shape, lambda i: tuple(0 for _ in shape))
  return pl.pallas_call(
      _tcb_body,
      grid=grid,
      in_specs=[
          pl.BlockSpec((NC, GBLK, E224), lambda i: (0, i, 0)),
          pl.BlockSpec((GBLK, G), lambda i: (i, 0)),
          full((1, E224)),
          full((8, G)), full((1, 8)),
          full((8, 8)), full((1, 8)),
          full((G, 8)), full((1, G)),
          full((128, E224)), full((128, G)), full((1, 128)),
          full((128, 128)), full((1, 128)),
          full((A, 128)), full((1, A)),
      ],
      out_specs=pl.BlockSpec((GBLK, A), lambda i: (i, 0)),
      out_shape=jax.ShapeDtypeStruct((B, A), F32),
  )(slabs, glob, brel2t, Wg1, bg1.reshape(1, 8), Wg2, bg2.reshape(1, 8),
    Wg3, bg3.reshape(1, G), wo1e, wo1g, bo1.reshape(1, 128),
    Wo2, bo2.reshape(1, 128), Wo3, bo3.reshape(1, A))


def kernel(x, edge_index, edge_attr, glob, Wrel1, brel1, Wroot1, Wrel2,
           brel2, Wroot2, Wg1, bg1, Wg2, bg2, Wg3, bg3, Wo1, bo1, Wo2, bo2,
           Wo3, bo3):
  src = edge_index[0]
  dst = edge_index[1]
  xbf = x.astype(jnp.bfloat16)
  agg1 = _sc_agg1(xbf, src, dst, edge_attr)
  pdup, qdup = _tc_layer1(xbf, agg1, Wrel1, brel1, Wroot1, Wrel2, Wroot2)
  agg2 = _sc_agg2(pdup, qdup, src, dst, edge_attr)
  slabs = agg2.reshape(NC, B, E224)
  return _tc_head(slabs, glob, brel2, Wg1, bg1, Wg2, bg2, Wg3, bg3,
                  Wo1, bo1, Wo2, bo2, Wo3, bo3)


# speedup vs baseline: 1.1693x; 1.0035x over previous
import functools

import jax
import jax.numpy as jnp
from jax import lax
from jax.experimental import pallas as pl
from jax.experimental.pallas import tpu as pltpu
from jax.experimental.pallas import tpu_sc as plsc

N = 55296
E = 442368
DIN = 256
DH = 512
B = 1024
G = 64
A = 18
NODES = 54

NC = 2
NS = 16
L = 16

F32 = jnp.float32
I32 = jnp.int32

C_PER_SC = 4
NCHUNK = NC * C_PER_SC
V = N // NCHUNK
SLAB_ROWS = V + L
EPT = E // NS
SB = 6912
NSB = EPT // SB
CAP_ROWS = SB // 128
RB = 128

MAGIC27 = 38837
MSHIFT = 20


def _iota16():
  return lax.iota(I32, L)


def _bi(s):
  return jnp.full((L,), s, I32)


def _sc1_body(x_hbm, src_hbm, dst_hbm, ew_hbm, agg_hbm,
              src_v, dst_v, ew_v, csrc, cdst, cew, rows_v, rows_w, slab,
              gsem, gsem2, ssemA, ssemB):
  core = lax.axis_index("c")
  tid = lax.axis_index("s")
  iota = _iota16()
  zeros32 = jnp.zeros((2 * L,), jnp.bfloat16)

  zrows_per_tile = V // NS
  for p in range(C_PER_SC):
    chunk = core * C_PER_SC + p
    lo = chunk * V

    def _z(r, _):
      for k in range(DIN // (2 * L)):
        rows_v[r, pl.ds(k * 2 * L, 2 * L)] = zeros32
      return 0
    lax.fori_loop(0, RB, _z, 0)

    zbase = pl.multiple_of(tid * zrows_per_tile, 16)
    nfull = zrows_per_tile // RB
    for zi in range(nfull):
      pltpu.sync_copy(rows_v, slab.at[pl.ds(zbase + zi * RB, RB)])
    rem = zrows_per_tile - nfull * RB
    if rem:
      pltpu.sync_copy(rows_v.at[pl.ds(0, rem)],
                      slab.at[pl.ds(zbase + nfull * RB, rem)])
    @pl.when(tid == 0)
    def _():
      pltpu.sync_copy(rows_v.at[pl.ds(0, L)], slab.at[pl.ds(V, L)])
    plsc.subcore_barrier()

    for sb in range(NSB):
      ebase = pl.multiple_of(tid * EPT + sb * SB, 128)
      pltpu.sync_copy(src_hbm.at[pl.ds(ebase, SB)], src_v)
      pltpu.sync_copy(dst_hbm.at[pl.ds(ebase, SB)], dst_v)
      pltpu.sync_copy(ew_hbm.at[pl.ds(ebase, SB)], ew_v)

      def _scan(i, cnt):
        off = i * L
        d = dst_v[pl.ds(off, L)]
        ld = d - _bi(lo)
        m = (ld >= _bi(0)) & (ld < _bi(V))
        cs = plsc.cumsum(m.astype(I32))
        pos = _bi(cnt) + cs - _bi(1)
        row = lax.shift_right_logical(pos, _bi(7))
        col = pos & _bi(127)
        plsc.store_scatter(cdst, [row, col], ld, mask=m)
        plsc.store_scatter(csrc, [row, col], src_v[pl.ds(off, L)], mask=m)
        plsc.store_scatter(cew, [row, col], ew_v[pl.ds(off, L)], mask=m)
        return cnt + jnp.sum(m.astype(I32))

      cnt = lax.fori_loop(0, SB // L, _scan, jnp.int32(0))
      nb = lax.shift_right_logical(cnt + 127, 7)
      lim = lax.shift_left(nb, 7)
      for k in range(RB // L):
        posp = _bi(cnt + k * L) + iota
        mp = posp < _bi(lim)
        rowp = lax.shift_right_logical(posp, _bi(7))
        colp = posp & _bi(127)
        plsc.store_scatter(cdst, [rowp, colp], _bi(V) + iota, mask=mp)
        plsc.store_scatter(csrc, [rowp, colp], iota, mask=mp)
        plsc.store_scatter(cew, [rowp, colp], jnp.zeros((L,), F32), mask=mp)

      def _scale(j, buf):
        jv = jnp.full((L,), j, I32)

        def _mul(r2, _):
          r = r2 * 2
          for dr in range(2):
            ewv = plsc.load_gather(cew, [jv, _bi(r + dr)])
            ew32 = plsc.pack(ewv, ewv, format=plsc.PackFormat.INTERLEAVED)
            for k in range(DIN // (2 * L)):
              sl = pl.ds(k * 2 * L, 2 * L)
              buf[r + dr, sl] = buf[r + dr, sl] * ew32
          return 0
        lax.fori_loop(0, RB // 2, _mul, 0)

      def _wait_scat(buf, sem):
        pltpu.make_async_copy(buf, slab.at[cdst.at[0]], sem).wait()

      @pl.when(nb > 0)
      def _():
        pltpu.async_copy(x_hbm.at[csrc.at[0]], rows_v, gsem)

      npair = lax.shift_right_logical(nb + 1, 1)

      def _pair(q, _):
        j0 = q * 2
        j1 = j0 + 1
        pltpu.make_async_copy(x_hbm.at[csrc.at[j0]], rows_v, gsem).wait()

        @pl.when(j1 < nb)
        def _():
          @pl.when(j0 > 0)
          def _():
            _wait_scat(rows_w, ssemB)
          pltpu.async_copy(x_hbm.at[csrc.at[j1]], rows_w, gsem2)
        _scale(j0, rows_v)
        pltpu.async_copy(rows_v, slab.at[cdst.at[j0]], ssemA, add=True)

        @pl.when(j1 < nb)
        def _():
          pltpu.make_async_copy(x_hbm.at[csrc.at[j1]], rows_w, gsem2).wait()

          @pl.when(j1 + 1 < nb)
          def _():
            _wait_scat(rows_v, ssemA)
            pltpu.async_copy(x_hbm.at[csrc.at[j1 + 1]], rows_v, gsem)
          _scale(j1, rows_w)
          pltpu.async_copy(rows_w, slab.at[cdst.at[j1]], ssemB, add=True)
        return 0

      lax.fori_loop(0, npair, _pair, 0)

      @pl.when(nb > 0)
      def _():
        lastA = ((nb - 1) & 1) == 0

        @pl.when(lastA)
        def _():
          _wait_scat(rows_v, ssemA)

        @pl.when(jnp.logical_not(lastA))
        def _():
          _wait_scat(rows_w, ssemB)

      @pl.when(nb > 1)
      def _():
        prevA = ((nb - 2) & 1) == 0

        @pl.when(prevA)
        def _():
          _wait_scat(rows_v, ssemA)

        @pl.when(jnp.logical_not(prevA))
        def _():
          _wait_scat(rows_w, ssemB)

    plsc.subcore_barrier()
    wrows = V // NS
    wbase = pl.multiple_of(tid * wrows, 16)
    pltpu.sync_copy(slab.at[pl.ds(wbase, wrows)],
                    agg_hbm.at[pl.ds(pl.multiple_of(lo + tid * wrows, 16),
                                     wrows)])
    plsc.subcore_barrier()


def _sc_agg1(xbf, src, dst, ew):
  mesh = plsc.VectorSubcoreMesh(core_axis_name="c", subcore_axis_name="s",
                                num_cores=NC, num_subcores=NS)
  f = pl.kernel(
      _sc1_body,
      out_type=jax.ShapeDtypeStruct((N, DIN), jnp.bfloat16),
      mesh=mesh,
      compiler_params=pltpu.CompilerParams(needs_layout_passes=False,
                                           use_tc_tiling_on_sc=False),
      scratch_types=[
          pltpu.VMEM((SB,), I32),
          pltpu.VMEM((SB,), I32),
          pltpu.VMEM((SB,), F32),
          pltpu.VMEM((CAP_ROWS, 128), I32),
          pltpu.VMEM((CAP_ROWS, 128), I32),
          pltpu.VMEM((CAP_ROWS, 128), F32),
          pltpu.VMEM((RB, DIN), jnp.bfloat16),
          pltpu.VMEM((RB, DIN), jnp.bfloat16),
          pltpu.VMEM_SHARED((SLAB_ROWS, DIN), jnp.bfloat16),
          pltpu.SemaphoreType.DMA,
          pltpu.SemaphoreType.DMA,
          pltpu.SemaphoreType.DMA,
          pltpu.SemaphoreType.DMA,
      ],
  )
  return f(xbf, src, dst, ew)


GROW = 28
SLAB2_ROWS = B * GROW
EPT2 = E // (NC * NS)
EB2 = 512
NB2 = EPT2 // EB2
NPT = N // (NC * NS)


def _sc2_body(pdup_hbm, qdup_hbm, src_hbm, dst_hbm, ew_hbm, out_hbm,
              srcb, dstb, ewb, rb, rows2, qbuf, qri, zbuf2, slab2, gsem):
  core = lax.axis_index("c")
  tid = lax.axis_index("s")
  w = core * NS + tid
  iota = _iota16()
  halfsel = lax.shift_right_logical(iota, _bi(2)) & _bi(1)
  qmask = ((halfsel == (lax.shift_right_logical(iota, _bi(3)) & _bi(1)))
           .astype(F32))

  def _z(r, _):
    flat = _bi(r * L) + iota
    plsc.store_scatter(zbuf2,
                       [lax.shift_right_logical(flat, _bi(3)), iota & _bi(7)],
                       jnp.zeros((L,), F32))
    return 0
  lax.fori_loop(0, 128, _z, 0)

  zbase = pl.multiple_of(tid * (SLAB2_ROWS // NS), 128)
  for zi in range(7):
    pltpu.sync_copy(zbuf2, slab2.at[pl.ds(zbase + zi * 256, 256)])
  plsc.subcore_barrier()

  nbase = w * NPT
  pltpu.sync_copy(qdup_hbm.at[pl.ds(pl.multiple_of(nbase, 64), NPT)], qbuf)

  def _qm(i, _):
    row = _bi(i * 2) + lax.shift_right_logical(iota, _bi(3))
    col = iota & _bi(7)
    v = plsc.load_gather(qbuf, [row, col])
    plsc.store_scatter(qbuf, [row, col], v * qmask)
    return 0

  def _qr(i, _):
    d = _bi(nbase + i * L) + iota
    t = lax.shift_right_logical(d, _bi(1))
    r = t + lax.shift_right_logical(t * _bi(MAGIC27), _bi(MSHIFT))
    qri[pl.ds(i * L, L)] = r
    return 0
  lax.fori_loop(0, (NPT * 8) // L, _qm, 0)
  lax.fori_loop(0, NPT // L, _qr, 0)
  pltpu.sync_copy(qbuf, slab2.at[qri], add=True)

  def _batch(j, _):
    ebase = pl.multiple_of(w * EPT2 + j * EB2, 128)
    pltpu.sync_copy(src_hbm.at[pl.ds(ebase, EB2)], srcb)
    pltpu.sync_copy(dst_hbm.at[pl.ds(ebase, EB2)], dstb)
    pltpu.sync_copy(ew_hbm.at[pl.ds(ebase, EB2)], ewb)
    pltpu.async_copy(pdup_hbm.at[srcb], rows2, gsem).wait()

    def _ri(i, _):
      d = dstb[pl.ds(i * L, L)]
      t = lax.shift_right_logical(d, _bi(1))
      rb[pl.ds(i * L, L)] = t + lax.shift_right_logical(
          t * _bi(MAGIC27), _bi(MSHIFT))
      return 0
    lax.fori_loop(0, EB2 // L, _ri, 0)

    def _mul(i, _):
      eidx = _bi(i * 2) + lax.shift_right_logical(iota, _bi(3))
      ewv = plsc.load_gather(ewb, [eidx])
      dv = plsc.load_gather(dstb, [eidx])
      keep = (halfsel == (dv & _bi(1))).astype(F32)
      rowi = eidx
      coli = iota & _bi(7)
      v = plsc.load_gather(rows2, [rowi, coli])
      plsc.store_scatter(rows2, [rowi, coli], v * ewv * keep)
      return 0
    lax.fori_loop(0, EB2 // 2, _mul, 0)
    pltpu.sync_copy(rows2, slab2.at[rb], add=True)
    return 0

  lax.fori_loop(0, NB2, _batch, 0)

  plsc.subcore_barrier()
  wrows = SLAB2_ROWS // NS
  wbase = pl.multiple_of(tid * wrows, 128)
  pltpu.sync_copy(slab2.at[pl.ds(wbase, wrows)],
                  out_hbm.at[core, pl.ds(wbase, wrows)])


def _sc_agg2(pdup, qdup, src, dst, ew):
  mesh = plsc.VectorSubcoreMesh(core_axis_name="c", subcore_axis_name="s",
                                num_cores=NC, num_subcores=NS)
  f = pl.kernel(
      _sc2_body,
      out_type=jax.ShapeDtypeStruct((NC, SLAB2_ROWS, 8), F32),
      mesh=mesh,
      compiler_params=pltpu.CompilerParams(needs_layout_passes=False,
                                           use_tc_tiling_on_sc=False),
      scratch_types=[
          pltpu.VMEM((EB2,), I32),
          pltpu.VMEM((EB2,), I32),
          pltpu.VMEM((EB2,), F32),
          pltpu.VMEM((EB2,), I32),
          pltpu.VMEM((EB2, 8), F32),
          pltpu.VMEM((NPT, 8), F32),
          pltpu.VMEM((NPT,), I32),
          pltpu.VMEM((256, 8), F32),
          pltpu.VMEM_SHARED((SLAB2_ROWS, 8), F32),
          pltpu.SemaphoreType.DMA,
      ],
  )
  return f(pdup, qdup, src, dst, ew)


RBA = 432


def _tca_body(x_ref, agg_ref, wrel1_ref, wroot1_ref, brel1_ref,
              wpd_ref, wqd_ref, pdup_ref, qdup_ref):
  dn = (((1,), (1,)), ((), ()))
  h = lax.dot_general(agg_ref[...], wrel1_ref[...], dn,
                      preferred_element_type=F32)
  h += lax.dot_general(x_ref[...], wroot1_ref[...], dn,
                       preferred_element_type=F32)
  h += brel1_ref[...]
  h = jnp.maximum(h, 0.0).astype(jnp.bfloat16)
  pdup_ref[...] = lax.dot_general(h, wpd_ref[...], dn,
                                  preferred_element_type=F32)
  qdup_ref[...] = lax.dot_general(h, wqd_ref[...], dn,
                                  preferred_element_type=F32)


def _tc_layer1(xbf, agg1, Wrel1, brel1, Wroot1, Wrel2, Wroot2):
  bf = jnp.bfloat16
  wpd = jnp.concatenate([Wrel2, Wrel2], axis=0).astype(bf)
  wqd = jnp.concatenate([Wroot2, Wroot2], axis=0).astype(bf)
  grid = (N // RBA,)
  return pl.pallas_call(
      _tca_body,
      grid=grid,
      in_specs=[
          pl.BlockSpec((RBA, DIN), lambda i: (i, 0)),
          pl.BlockSpec((RBA, DIN), lambda i: (i, 0)),
          pl.BlockSpec((DH, DIN), lambda i: (0, 0)),
          pl.BlockSpec((DH, DIN), lambda i: (0, 0)),
          pl.BlockSpec((1, DH), lambda i: (0, 0)),
          pl.BlockSpec((8, DH), lambda i: (0, 0)),
          pl.BlockSpec((8, DH), lambda i: (0, 0)),
      ],
      out_specs=[
          pl.BlockSpec((RBA, 8), lambda i: (i, 0)),
          pl.BlockSpec((RBA, 8), lambda i: (i, 0)),
      ],
      out_shape=[
          jax.ShapeDtypeStruct((N, 8), F32),
          jax.ShapeDtypeStruct((N, 8), F32),
      ],
  )(xbf, agg1, Wrel1.astype(bf), Wroot1.astype(bf),
    brel1.reshape(1, DH), wpd, wqd)


GBLK = 256
E224 = GROW * 8


def _tcb_body(s_ref, glob_ref, brel2t_ref, wg1_ref, bg1_ref, wg2_ref,
              bg2_ref, wg3_ref, bg3_ref, wo1e_ref, wo1g_ref, bo1_ref,
              wo2_ref, bo2_ref, wo3_ref, bo3_ref, out_ref):
  dn = (((1,), (1,)), ((), ()))
  e = jnp.maximum(s_ref[0] + s_ref[1] + brel2t_ref[...], 0.0)
  g = jnp.maximum(lax.dot_general(glob_ref[...], wg1_ref[...], dn,
                                  preferred_element_type=F32)
                  + bg1_ref[...], 0.0)
  g = jnp.maximum(lax.dot_general(g, wg2_ref[...], dn,
                                  preferred_element_type=F32)
                  + bg2_ref[...], 0.0)
  g = jnp.maximum(lax.dot_general(g, wg3_ref[...], dn,
                                  preferred_element_type=F32)
                  + bg3_ref[...], 0.0)
  o = lax.dot_general(e, wo1e_ref[...], dn, preferred_element_type=F32)
  o += lax.dot_general(g, wo1g_ref[...], dn, preferred_element_type=F32)
  o = jnp.maximum(o + bo1_ref[...], 0.0)
  o = jnp.maximum(lax.dot_general(o, wo2_ref[...], dn,
                                  preferred_element_type=F32)
                  + bo2_ref[...], 0.0)
  out_ref[...] = (lax.dot_general(o, wo3_ref[...], dn,
                                  preferred_element_type=F32)
                  + bo3_ref[...])


def _tc_head(slabs, glob, brel2, Wg1, bg1, Wg2, bg2, Wg3, bg3,
             Wo1, bo1, Wo2, bo2, Wo3, bo3):
  brel2t = jnp.concatenate(
      [jnp.tile(brel2, NODES), jnp.zeros((8,), F32)]).reshape(1, E224)
  wo1e = jnp.concatenate(
      [Wo1[:, :NODES * 4], jnp.zeros((128, 8), F32)], axis=1)
  wo1g = Wo1[:, NODES * 4:]
  grid = (B // GBLK,)
  full = lambda shape: pl.BlockSpec(shape, lambda i: tuple(0 for _ in shape))
  return pl.pallas_call(
      _tcb_body,
      grid=grid,
      in_specs=[
          pl.BlockSpec((NC, GBLK, E224), lambda i: (0, i, 0)),
          pl.BlockSpec((GBLK, G), lambda i: (i, 0)),
          full((1, E224)),
          full((8, G)), full((1, 8)),
          full((8, 8)), full((1, 8)),
          full((G, 8)), full((1, G)),
          full((128, E224)), full((128, G)), full((1, 128)),
          full((128, 128)), full((1, 128)),
          full((A, 128)), full((1, A)),
      ],
      out_specs=pl.BlockSpec((GBLK, A), lambda i: (i, 0)),
      out_shape=jax.ShapeDtypeStruct((B, A), F32),
  )(slabs, glob, brel2t, Wg1, bg1.reshape(1, 8), Wg2, bg2.reshape(1, 8),
    Wg3, bg3.reshape(1, G), wo1e, wo1g, bo1.reshape(1, 128),
    Wo2, bo2.reshape(1, 128), Wo3, bo3.reshape(1, A))


def kernel(x, edge_index, edge_attr, glob, Wrel1, brel1, Wroot1, Wrel2,
           brel2, Wroot2, Wg1, bg1, Wg2, bg2, Wg3, bg3, Wo1, bo1, Wo2, bo2,
           Wo3, bo3):
  src = edge_index[0]
  dst = edge_index[1]
  xbf = x.astype(jnp.bfloat16)
  agg1 = _sc_agg1(xbf, src, dst, edge_attr)
  pdup, qdup = _tc_layer1(xbf, agg1, Wrel1, brel1, Wroot1, Wrel2, Wroot2)
  agg2 = _sc_agg2(pdup, qdup, src, dst, edge_attr)
  slabs = agg2.reshape(NC, B, E224)
  return _tc_head(slabs, glob, brel2, Wg1, bg1, Wg2, bg2, Wg3, bg3,
                  Wo1, bo1, Wo2, bo2, Wo3, bo3)
